# Initial kernel scaffold; baseline (speedup 1.0000x reference)
import jax
import jax.numpy as jnp
from jax.experimental import pallas as pl


def kernel(t, pe):
    raise NotImplementedError("write your pallas kernel here")


# speedup vs baseline: 2.1800x; 2.1800x over previous
import functools

import jax
import jax.numpy as jnp
from jax import lax
from jax.experimental import pallas as pl
from jax.experimental.pallas import tpu as pltpu
from jax.experimental.pallas import tpu_sc as plsc

D_MODEL = 1024
N_IDX = 4 * 8192

_info = plsc.get_sparse_core_info()
NC, NS = _info.num_cores, _info.num_subcores
NW = NC * NS
B_PER_W = N_IDX // NW
CHUNK = 64
N_CHUNK = B_PER_W // CHUNK


def _gather_body(t_hbm, pe_hbm, out_hbm, idx_v, rows_v, sem):
    wid = lax.axis_index("s") * NC + lax.axis_index("c")
    base = wid * B_PER_W
    pltpu.sync_copy(t_hbm.at[pl.ds(base, B_PER_W)], idx_v)

    def chunk_step(g, carry):
        off = g * CHUNK
        pltpu.async_copy(pe_hbm.at[idx_v.at[pl.ds(off, CHUNK)]], rows_v, sem).wait()
        pltpu.sync_copy(rows_v, out_hbm.at[pl.ds(base + off, CHUNK)])
        return carry

    lax.fori_loop(0, N_CHUNK, chunk_step, 0)


@functools.partial(jax.jit, static_argnames=())
def kernel(t, pe):
    t_flat = t.reshape(-1)
    grid_kernel = functools.partial(
        pl.kernel,
        mesh=plsc.VectorSubcoreMesh(core_axis_name="c", subcore_axis_name="s"),
        out_type=jax.ShapeDtypeStruct((N_IDX, D_MODEL), jnp.float32),
        scratch_types=[
            pltpu.VMEM((B_PER_W,), jnp.int32),
            pltpu.VMEM((CHUNK, D_MODEL), jnp.float32),
            pltpu.SemaphoreType.DMA,
        ],
    )
    out = grid_kernel(_gather_body)(t_flat, pe)
    return out.reshape(t.shape + (D_MODEL,))


# speedup vs baseline: 2.3728x; 1.0885x over previous
import functools

import jax
import jax.numpy as jnp
from jax import lax
from jax.experimental import pallas as pl
from jax.experimental.pallas import tpu as pltpu
from jax.experimental.pallas import tpu_sc as plsc

D_MODEL = 1024
N_IDX = 4 * 8192

_info = plsc.get_sparse_core_info()
NC, NS = _info.num_cores, _info.num_subcores
NW = NC * NS
B_PER_W = N_IDX // NW
CHUNK = 32
NBUF = 2
N_CHUNK = B_PER_W // CHUNK


def _gather_body(t_hbm, pe_hbm, out_hbm, idx_v, buf0, buf1, sem0, sem1):
    wid = lax.axis_index("s") * NC + lax.axis_index("c")
    base = wid * B_PER_W
    pltpu.sync_copy(t_hbm.at[pl.ds(base, B_PER_W)], idx_v)
    bufs = (buf0, buf1)
    sems = (sem0, sem1)

    for b in range(NBUF):
        pltpu.async_copy(
            pe_hbm.at[idx_v.at[pl.ds(b * CHUNK, CHUNK)]], bufs[b], sems[b])

    def step(i, carry):
        for b in range(NBUF):
            off = (i * NBUF + b) * CHUNK
            pltpu.make_async_copy(
                pe_hbm.at[pl.ds(0, CHUNK)], bufs[b], sems[b]).wait()
            pltpu.sync_copy(bufs[b], out_hbm.at[pl.ds(base + off, CHUNK)])
            pltpu.async_copy(
                pe_hbm.at[idx_v.at[pl.ds(off + NBUF * CHUNK, CHUNK)]],
                bufs[b], sems[b])
        return carry

    lax.fori_loop(0, N_CHUNK // NBUF - 1, step, 0)

    for b in range(NBUF):
        off = (N_CHUNK - NBUF + b) * CHUNK
        pltpu.make_async_copy(
            pe_hbm.at[pl.ds(0, CHUNK)], bufs[b], sems[b]).wait()
        pltpu.sync_copy(bufs[b], out_hbm.at[pl.ds(base + off, CHUNK)])


@jax.jit
def kernel(t, pe):
    t_flat = t.reshape(-1)
    grid_kernel = functools.partial(
        pl.kernel,
        mesh=plsc.VectorSubcoreMesh(core_axis_name="c", subcore_axis_name="s"),
        out_type=jax.ShapeDtypeStruct((N_IDX, D_MODEL), jnp.float32),
        scratch_types=[
            pltpu.VMEM((B_PER_W,), jnp.int32),
            pltpu.VMEM((CHUNK, D_MODEL), jnp.float32),
            pltpu.VMEM((CHUNK, D_MODEL), jnp.float32),
            pltpu.SemaphoreType.DMA,
            pltpu.SemaphoreType.DMA,
        ],
    )
    out = grid_kernel(_gather_body)(t_flat, pe)
    return out.reshape(t.shape + (D_MODEL,))


# speedup vs baseline: 2.3732x; 1.0002x over previous
import functools

import jax
import jax.numpy as jnp
from jax import lax
from jax.experimental import pallas as pl
from jax.experimental.pallas import tpu as pltpu
from jax.experimental.pallas import tpu_sc as plsc

D_MODEL = 1024
N_IDX = 4 * 8192

_info = plsc.get_sparse_core_info()
NC, NS = _info.num_cores, _info.num_subcores
NW = NC * NS
B_PER_W = N_IDX // NW
CHUNK = 16
NBUF = 4
N_CHUNK = B_PER_W // CHUNK
assert N_CHUNK % NBUF == 0


def _gather_body(t_hbm, pe_hbm, out_hbm, idx_v, *rest):
    bufs = rest[:NBUF]
    sems = rest[NBUF:]
    wid = lax.axis_index("s") * NC + lax.axis_index("c")
    base = wid * B_PER_W
    pltpu.sync_copy(t_hbm.at[pl.ds(base, B_PER_W)], idx_v)

    for b in range(NBUF):
        pltpu.async_copy(
            pe_hbm.at[idx_v.at[pl.ds(b * CHUNK, CHUNK)]], bufs[b], sems[b])

    def step(i, carry):
        for b in range(NBUF):
            off = (i * NBUF + b) * CHUNK
            pltpu.make_async_copy(
                pe_hbm.at[pl.ds(0, CHUNK)], bufs[b], sems[b]).wait()
            pltpu.sync_copy(bufs[b], out_hbm.at[pl.ds(base + off, CHUNK)])
            pltpu.async_copy(
                pe_hbm.at[idx_v.at[pl.ds(off + NBUF * CHUNK, CHUNK)]],
                bufs[b], sems[b])
        return carry

    lax.fori_loop(0, N_CHUNK // NBUF - 1, step, 0)

    for b in range(NBUF):
        off = (N_CHUNK - NBUF + b) * CHUNK
        pltpu.make_async_copy(
            pe_hbm.at[pl.ds(0, CHUNK)], bufs[b], sems[b]).wait()
        pltpu.sync_copy(bufs[b], out_hbm.at[pl.ds(base + off, CHUNK)])


@jax.jit
def kernel(t, pe):
    t_flat = t.reshape(-1)
    grid_kernel = functools.partial(
        pl.kernel,
        mesh=plsc.VectorSubcoreMesh(core_axis_name="c", subcore_axis_name="s"),
        out_type=jax.ShapeDtypeStruct((N_IDX, D_MODEL), jnp.float32),
        scratch_types=(
            [pltpu.VMEM((B_PER_W,), jnp.int32)]
            + [pltpu.VMEM((CHUNK, D_MODEL), jnp.float32)] * NBUF
            + [pltpu.SemaphoreType.DMA] * NBUF
        ),
    )
    out = grid_kernel(_gather_body)(t_flat, pe)
    return out.reshape(t.shape + (D_MODEL,))
